# Initial kernel scaffold; baseline (speedup 1.0000x reference)
import jax
import jax.numpy as jnp
from jax.experimental import pallas as pl


def kernel(p1, x1, o1, p2, x2, o2, W, b, gamma, beta):
    raise NotImplementedError("write your pallas kernel here")


# speedup vs baseline: 12.1745x; 12.1745x over previous
import functools

import jax
import jax.numpy as jnp
from jax import lax
from jax.experimental import pallas as pl
from jax.experimental.pallas import tpu as pltpu
from jax.experimental.pallas import tpu_sc as plsc

N1 = 8192
N2 = 2048
D_DENSE = 128
D_SPARSE = 256
D_OUT = 256
K = 3

BLK = 512
NBLK = N1 // BLK
LANES = 16
NW = 32
BPW = N1 // NW
CH = 64
NCH = BPW // CH


def _knn_body(p1_ref, p2t_ref, i1_ref, i2_ref, i3_ref, w1_ref, w2_ref, w3_ref):
    x = p1_ref[...]
    pt = p2t_ref[...]
    d2 = jnp.zeros((BLK, N2), jnp.float32)
    for c in range(3):
        diff = x[:, c : c + 1] - pt[c : c + 1, :]
        d2 = d2 + diff * diff
    cols = lax.broadcasted_iota(jnp.int32, (BLK, N2), 1)
    work = d2
    idxs, dists = [], []
    for _ in range(K):
        m = jnp.min(work, axis=1, keepdims=True)
        sel = jnp.where(work <= m, cols, N2)
        ik = jnp.min(sel, axis=1, keepdims=True)
        idxs.append(ik)
        dists.append(m)
        work = jnp.where(cols == ik, 1e30, work)
    recips = [1.0 / (jnp.sqrt(jnp.maximum(m, 1e-12)) + 1e-8) for m in dists]
    norm = recips[0] + recips[1] + recips[2]
    for ref, ik in zip((i1_ref, i2_ref, i3_ref), idxs):
        ref[...] = ik
    for ref, r in zip((w1_ref, w2_ref, w3_ref), recips):
        ref[...] = r / norm


def _tc_knn(p1pad, p2t):
    out_i = jax.ShapeDtypeStruct((N1, 1), jnp.int32)
    out_w = jax.ShapeDtypeStruct((N1, 1), jnp.float32)
    return pl.pallas_call(
        _knn_body,
        grid=(NBLK,),
        in_specs=[
            pl.BlockSpec((BLK, 8), lambda i: (i, 0)),
            pl.BlockSpec((8, N2), lambda i: (0, 0)),
        ],
        out_specs=[pl.BlockSpec((BLK, 1), lambda i: (i, 0))] * 6,
        out_shape=[out_i] * 3 + [out_w] * 3,
    )(p1pad, p2t)


def _sc_body(i1, i2, i3, w1, w2, w3, x2_hbm, out_hbm,
             i1v, i2v, i3v, w1v, w2v, w3v, rA, rB, rC, outv, sem):
    wid = lax.axis_index("s") * 2 + lax.axis_index("c")
    base = wid * BPW
    pltpu.sync_copy(i1.at[pl.ds(base, BPW)], i1v)
    pltpu.sync_copy(i2.at[pl.ds(base, BPW)], i2v)
    pltpu.sync_copy(i3.at[pl.ds(base, BPW)], i3v)
    pltpu.sync_copy(w1.at[pl.ds(base, BPW)], w1v)
    pltpu.sync_copy(w2.at[pl.ds(base, BPW)], w2v)
    pltpu.sync_copy(w3.at[pl.ds(base, BPW)], w3v)
    for c in range(NCH):
        q0 = c * CH
        cpA = pltpu.async_copy(x2_hbm.at[i1v.at[pl.ds(q0, CH)]], rA, sem)
        cpB = pltpu.async_copy(x2_hbm.at[i2v.at[pl.ds(q0, CH)]], rB, sem)
        cpC = pltpu.async_copy(x2_hbm.at[i3v.at[pl.ds(q0, CH)]], rC, sem)
        cpA.wait()
        cpB.wait()
        cpC.wait()

        def body(q, _):
            qi = jnp.full((LANES,), q0 + q, jnp.int32)
            wa = plsc.load_gather(w1v, [qi])
            wb = plsc.load_gather(w2v, [qi])
            wc = plsc.load_gather(w3v, [qi])
            for f in range(D_SPARSE // LANES):
                s = pl.ds(f * LANES, LANES)
                outv[q, s] = (wa * rA[q, s] + wb * rB[q, s] + wc * rC[q, s])
            return 0

        lax.fori_loop(0, CH, body, 0)
        pltpu.sync_copy(outv, out_hbm.at[pl.ds(base + q0, CH)])


def _sc_gather(i1, i2, i3, w1, w2, w3, x2):
    mesh = plsc.VectorSubcoreMesh(core_axis_name="c", subcore_axis_name="s")
    fn = pl.kernel(
        _sc_body,
        out_type=jax.ShapeDtypeStruct((N1, D_SPARSE), jnp.float32),
        mesh=mesh,
        compiler_params=pltpu.CompilerParams(needs_layout_passes=False),
        scratch_types=(
            [pltpu.VMEM((BPW,), jnp.int32)] * 3
            + [pltpu.VMEM((BPW,), jnp.float32)] * 3
            + [pltpu.VMEM((CH, D_SPARSE), jnp.float32)] * 4
            + [pltpu.SemaphoreType.DMA]
        ),
    )
    return fn(i1, i2, i3, w1, w2, w3, x2)


def _mlp_body(x1_ref, it_ref, w1t_ref, w2t_ref, b_ref, y_ref, ps_ref, pss_ref):
    y = (
        jnp.dot(x1_ref[...], w1t_ref[...], preferred_element_type=jnp.float32)
        + jnp.dot(it_ref[...], w2t_ref[...], preferred_element_type=jnp.float32)
        + b_ref[...]
    )
    y_ref[...] = y
    ps_ref[...] = jnp.sum(y, axis=0, keepdims=True).reshape(1, 1, D_OUT)
    pss_ref[...] = jnp.sum(y * y, axis=0, keepdims=True).reshape(1, 1, D_OUT)


def _tc_mlp(x1, interp, w1t, w2t, b2d):
    return pl.pallas_call(
        _mlp_body,
        grid=(NBLK,),
        in_specs=[
            pl.BlockSpec((BLK, D_DENSE), lambda i: (i, 0)),
            pl.BlockSpec((BLK, D_SPARSE), lambda i: (i, 0)),
            pl.BlockSpec((D_DENSE, D_OUT), lambda i: (0, 0)),
            pl.BlockSpec((D_SPARSE, D_OUT), lambda i: (0, 0)),
            pl.BlockSpec((1, D_OUT), lambda i: (0, 0)),
        ],
        out_specs=[
            pl.BlockSpec((BLK, D_OUT), lambda i: (i, 0)),
            pl.BlockSpec((1, 1, D_OUT), lambda i: (i, 0, 0)),
            pl.BlockSpec((1, 1, D_OUT), lambda i: (i, 0, 0)),
        ],
        out_shape=[
            jax.ShapeDtypeStruct((N1, D_OUT), jnp.float32),
            jax.ShapeDtypeStruct((NBLK, 1, D_OUT), jnp.float32),
            jax.ShapeDtypeStruct((NBLK, 1, D_OUT), jnp.float32),
        ],
    )(x1, interp, w1t, w2t, b2d)


def _bn_body(y_ref, ps_ref, pss_ref, g_ref, be_ref, o_ref):
    inv_n = 1.0 / N1
    mean = jnp.sum(ps_ref[...], axis=0, keepdims=True) * inv_n
    ex2 = jnp.sum(pss_ref[...], axis=0, keepdims=True) * inv_n
    var = ex2 - mean * mean
    scale = lax.rsqrt(var + 1e-5) * g_ref[...]
    o_ref[...] = jnp.maximum((y_ref[...] - mean) * scale + be_ref[...], 0.0)


def _tc_bn(y, ps, pss, g2d, be2d):
    return pl.pallas_call(
        _bn_body,
        grid=(NBLK,),
        in_specs=[
            pl.BlockSpec((BLK, D_OUT), lambda i: (i, 0)),
            pl.BlockSpec((NBLK, D_OUT), lambda i: (0, 0)),
            pl.BlockSpec((NBLK, D_OUT), lambda i: (0, 0)),
            pl.BlockSpec((1, D_OUT), lambda i: (0, 0)),
            pl.BlockSpec((1, D_OUT), lambda i: (0, 0)),
        ],
        out_specs=pl.BlockSpec((BLK, D_OUT), lambda i: (i, 0)),
        out_shape=jax.ShapeDtypeStruct((N1, D_OUT), jnp.float32),
    )(y, ps, pss, g2d, be2d)


def kernel(p1, x1, o1, p2, x2, o2, W, b, gamma, beta):
    p1pad = jnp.zeros((N1, 8), jnp.float32).at[:, :3].set(p1)
    p2t = jnp.zeros((8, N2), jnp.float32).at[:3, :].set(p2.T)
    i1, i2, i3, w1, w2, w3 = _tc_knn(p1pad, p2t)
    interp = _sc_gather(
        i1.reshape(N1), i2.reshape(N1), i3.reshape(N1),
        w1.reshape(N1), w2.reshape(N1), w3.reshape(N1), x2
    )
    w1t = W[:, :D_DENSE].T
    w2t = W[:, D_DENSE:].T
    y, ps, pss = _tc_mlp(x1, interp, w1t, w2t, b.reshape(1, D_OUT))
    out = _tc_bn(y, ps.reshape(NBLK, D_OUT), pss.reshape(NBLK, D_OUT),
                 gamma.reshape(1, D_OUT), beta.reshape(1, D_OUT))
    return (p1, out, o1)


# speedup vs baseline: 12.8868x; 1.0585x over previous
import functools

import jax
import jax.numpy as jnp
from jax import lax
from jax.experimental import pallas as pl
from jax.experimental.pallas import tpu as pltpu
from jax.experimental.pallas import tpu_sc as plsc

N1 = 8192
N2 = 2048
D_DENSE = 128
D_SPARSE = 256
D_OUT = 256
K = 3

BLK = 512
NBLK = N1 // BLK
LANES = 16
NW = 32
BPW = N1 // NW
CH = 64
NCH = BPW // CH


def _knn_body(p1_ref, p2t_ref, i1_ref, i2_ref, i3_ref, w1_ref, w2_ref, w3_ref):
    x = p1_ref[...]
    pt = p2t_ref[...]
    d2 = jnp.zeros((BLK, N2), jnp.float32)
    for c in range(3):
        diff = x[:, c : c + 1] - pt[c : c + 1, :]
        d2 = d2 + diff * diff
    colsf = lax.broadcasted_iota(jnp.int32, (BLK, N2), 1).astype(jnp.float32)
    work = d2
    idxs, dists = [], []
    for _ in range(K):
        m = jnp.min(work, axis=1, keepdims=True)
        sel = jnp.where(work <= m, colsf, float(N2))
        ikf = jnp.min(sel, axis=1, keepdims=True)
        idxs.append(ikf.astype(jnp.int32))
        dists.append(m)
        work = jnp.where(colsf == ikf, 1e30, work)
    recips = [1.0 / (jnp.sqrt(jnp.maximum(m, 1e-12)) + 1e-8) for m in dists]
    norm = recips[0] + recips[1] + recips[2]
    for ref, ik in zip((i1_ref, i2_ref, i3_ref), idxs):
        ref[...] = ik
    for ref, r in zip((w1_ref, w2_ref, w3_ref), recips):
        ref[...] = r / norm


def _tc_knn(p1pad, p2t):
    out_i = jax.ShapeDtypeStruct((N1, 1), jnp.int32)
    out_w = jax.ShapeDtypeStruct((N1, 1), jnp.float32)
    return pl.pallas_call(
        _knn_body,
        grid=(NBLK,),
        in_specs=[
            pl.BlockSpec((BLK, 8), lambda i: (i, 0)),
            pl.BlockSpec((8, N2), lambda i: (0, 0)),
        ],
        out_specs=[pl.BlockSpec((BLK, 1), lambda i: (i, 0))] * 6,
        out_shape=[out_i] * 3 + [out_w] * 3,
    )(p1pad, p2t)


def _sc_body(i1, i2, i3, w1, w2, w3, x2_hbm, out_hbm,
             i1v, i2v, i3v, w1v, w2v, w3v, rA, rB, rC, outv, sem):
    wid = lax.axis_index("s") * 2 + lax.axis_index("c")
    base = wid * BPW
    pltpu.sync_copy(i1.at[pl.ds(base, BPW)], i1v)
    pltpu.sync_copy(i2.at[pl.ds(base, BPW)], i2v)
    pltpu.sync_copy(i3.at[pl.ds(base, BPW)], i3v)
    pltpu.sync_copy(w1.at[pl.ds(base, BPW)], w1v)
    pltpu.sync_copy(w2.at[pl.ds(base, BPW)], w2v)
    pltpu.sync_copy(w3.at[pl.ds(base, BPW)], w3v)
    for c in range(NCH):
        q0 = c * CH
        cpA = pltpu.async_copy(x2_hbm.at[i1v.at[pl.ds(q0, CH)]], rA, sem)
        cpB = pltpu.async_copy(x2_hbm.at[i2v.at[pl.ds(q0, CH)]], rB, sem)
        cpC = pltpu.async_copy(x2_hbm.at[i3v.at[pl.ds(q0, CH)]], rC, sem)
        cpA.wait()
        cpB.wait()
        cpC.wait()

        def body(q, _):
            qi = jnp.full((LANES,), q0 + q, jnp.int32)
            wa = plsc.load_gather(w1v, [qi])
            wb = plsc.load_gather(w2v, [qi])
            wc = plsc.load_gather(w3v, [qi])
            for f in range(D_SPARSE // LANES):
                s = pl.ds(f * LANES, LANES)
                outv[q, s] = (wa * rA[q, s] + wb * rB[q, s] + wc * rC[q, s])
            return 0

        lax.fori_loop(0, CH, body, 0)
        pltpu.sync_copy(outv, out_hbm.at[pl.ds(base + q0, CH)])


def _sc_gather(i1, i2, i3, w1, w2, w3, x2):
    mesh = plsc.VectorSubcoreMesh(core_axis_name="c", subcore_axis_name="s")
    fn = pl.kernel(
        _sc_body,
        out_type=jax.ShapeDtypeStruct((N1, D_SPARSE), jnp.float32),
        mesh=mesh,
        compiler_params=pltpu.CompilerParams(needs_layout_passes=False),
        scratch_types=(
            [pltpu.VMEM((BPW,), jnp.int32)] * 3
            + [pltpu.VMEM((BPW,), jnp.float32)] * 3
            + [pltpu.VMEM((CH, D_SPARSE), jnp.float32)] * 4
            + [pltpu.SemaphoreType.DMA]
        ),
    )
    return fn(i1, i2, i3, w1, w2, w3, x2)


def _mlp_body(x1_ref, it_ref, w1t_ref, w2t_ref, b_ref, y_ref, ps_ref, pss_ref):
    y = (
        jnp.dot(x1_ref[...], w1t_ref[...], preferred_element_type=jnp.float32)
        + jnp.dot(it_ref[...], w2t_ref[...], preferred_element_type=jnp.float32)
        + b_ref[...]
    )
    y_ref[...] = y
    ps_ref[...] = jnp.sum(y, axis=0, keepdims=True).reshape(1, 1, D_OUT)
    pss_ref[...] = jnp.sum(y * y, axis=0, keepdims=True).reshape(1, 1, D_OUT)


def _tc_mlp(x1, interp, w1t, w2t, b2d):
    return pl.pallas_call(
        _mlp_body,
        grid=(NBLK,),
        in_specs=[
            pl.BlockSpec((BLK, D_DENSE), lambda i: (i, 0)),
            pl.BlockSpec((BLK, D_SPARSE), lambda i: (i, 0)),
            pl.BlockSpec((D_DENSE, D_OUT), lambda i: (0, 0)),
            pl.BlockSpec((D_SPARSE, D_OUT), lambda i: (0, 0)),
            pl.BlockSpec((1, D_OUT), lambda i: (0, 0)),
        ],
        out_specs=[
            pl.BlockSpec((BLK, D_OUT), lambda i: (i, 0)),
            pl.BlockSpec((1, 1, D_OUT), lambda i: (i, 0, 0)),
            pl.BlockSpec((1, 1, D_OUT), lambda i: (i, 0, 0)),
        ],
        out_shape=[
            jax.ShapeDtypeStruct((N1, D_OUT), jnp.float32),
            jax.ShapeDtypeStruct((NBLK, 1, D_OUT), jnp.float32),
            jax.ShapeDtypeStruct((NBLK, 1, D_OUT), jnp.float32),
        ],
    )(x1, interp, w1t, w2t, b2d)


def _bn_body(y_ref, ps_ref, pss_ref, g_ref, be_ref, o_ref):
    inv_n = 1.0 / N1
    mean = jnp.sum(ps_ref[...], axis=0, keepdims=True) * inv_n
    ex2 = jnp.sum(pss_ref[...], axis=0, keepdims=True) * inv_n
    var = ex2 - mean * mean
    scale = lax.rsqrt(var + 1e-5) * g_ref[...]
    o_ref[...] = jnp.maximum((y_ref[...] - mean) * scale + be_ref[...], 0.0)


def _tc_bn(y, ps, pss, g2d, be2d):
    return pl.pallas_call(
        _bn_body,
        grid=(NBLK,),
        in_specs=[
            pl.BlockSpec((BLK, D_OUT), lambda i: (i, 0)),
            pl.BlockSpec((NBLK, D_OUT), lambda i: (0, 0)),
            pl.BlockSpec((NBLK, D_OUT), lambda i: (0, 0)),
            pl.BlockSpec((1, D_OUT), lambda i: (0, 0)),
            pl.BlockSpec((1, D_OUT), lambda i: (0, 0)),
        ],
        out_specs=pl.BlockSpec((BLK, D_OUT), lambda i: (i, 0)),
        out_shape=jax.ShapeDtypeStruct((N1, D_OUT), jnp.float32),
    )(y, ps, pss, g2d, be2d)


def kernel(p1, x1, o1, p2, x2, o2, W, b, gamma, beta):
    p1pad = jnp.zeros((N1, 8), jnp.float32).at[:, :3].set(p1)
    p2t = jnp.zeros((8, N2), jnp.float32).at[:3, :].set(p2.T)
    i1, i2, i3, w1, w2, w3 = _tc_knn(p1pad, p2t)
    interp = _sc_gather(
        i1.reshape(N1), i2.reshape(N1), i3.reshape(N1),
        w1.reshape(N1), w2.reshape(N1), w3.reshape(N1), x2
    )
    w1t = W[:, :D_DENSE].T
    w2t = W[:, D_DENSE:].T
    y, ps, pss = _tc_mlp(x1, interp, w1t, w2t, b.reshape(1, D_OUT))
    out = _tc_bn(y, ps.reshape(NBLK, D_OUT), pss.reshape(NBLK, D_OUT),
                 gamma.reshape(1, D_OUT), beta.reshape(1, D_OUT))
    return (p1, out, o1)
